# baseline (speedup 1.0000x reference)
import functools

import jax
import jax.numpy as jnp
from jax import lax
from jax.experimental import pallas as pl
from jax.experimental.pallas import tpu as pltpu
from jax.experimental.pallas import tpu_sc as plsc

_EPS = 1e-06
_N = 50000
_D = 256
_NSEG = 128
_C = 112
_NCHUNK = (_N + _C - 1) // _C
_L = 16
_NW = 32
_JD = _D // _L


def _sc_partial_kernel(feat_hbm, ids_hbm, out_hbm, chunk_v, idx_v, acc_v,
                       semf0, semf1, semi0, semi1):
    cid = lax.axis_index("c")
    sid = lax.axis_index("s")
    wid = sid * 2 + cid
    semf = (semf0, semf1)
    semi = (semi0, semi1)

    zvec = jnp.zeros((_L,), jnp.float32)
    lane = lax.iota(jnp.int32, _L)

    def zero_row(r, carry):
        for j in range(_JD):
            acc_v[pl.ds(r * _D + j * _L, _L)] = zvec
        return carry

    lax.fori_loop(0, 2 * _NSEG, zero_row, 0)

    c_lo = (wid * _NCHUNK) // _NW
    c_hi = ((wid + 1) * _NCHUNK) // _NW

    def start(c, b):
        off = lax.min(c * _C, _N - _C)
        pltpu.async_copy(feat_hbm.at[pl.ds(off, _C)], chunk_v.at[b], semf[b])
        pltpu.async_copy(ids_hbm.at[pl.ds(off, _C)], idx_v.at[b], semi[b])

    def process(c, b):
        off = lax.min(c * _C, _N - _C)
        nzero = c * _C - off
        pltpu.make_async_copy(feat_hbm.at[pl.ds(off, _C)], chunk_v.at[b],
                              semf[b]).wait()
        pltpu.make_async_copy(ids_hbm.at[pl.ds(off, _C)], idx_v.at[b],
                              semi[b]).wait()

        def zrow(r, carry2):
            for j in range(_JD):
                chunk_v[b, r, pl.ds(j * _L, _L)] = zvec
            return carry2

        lax.fori_loop(0, nzero, zrow, 0)

        def group_body(g, carry2):
            seg_vec = idx_v[b, pl.ds(g * _L, _L)]
            seg0 = seg_vec.at[jnp.zeros((_L,), jnp.int32)].get(
                mode="promise_in_bounds")
            uniform = jnp.all(seg_vec == seg0)
            row0 = g * _L

            def fast(_):
                def racc(r, carry):
                    out = []
                    for j in range(_JD):
                        x = chunk_v[b, row0 + r, pl.ds(j * _L, _L)]
                        out.append(carry[j] + x)
                        out.append(carry[_JD + j] + x * x)
                    return tuple(out[0::2]) + tuple(out[1::2])

                acc = lax.fori_loop(0, _L, racc, (zvec,) * (2 * _JD))
                base = seg0 * _D + lane
                for j in range(_JD):
                    plsc.addupdate_scatter(acc_v, [base + (j * _L)], acc[j])
                    plsc.addupdate_scatter(
                        acc_v, [base + (_NSEG * _D + j * _L)], acc[_JD + j])
                return 0

            def slow(_):
                def row_body(r, carry3):
                    ridx = jnp.full((_L,), r, jnp.int32)
                    seg = seg_vec.at[ridx].get(mode="promise_in_bounds")
                    base = seg * _D + lane
                    row = row0 + r
                    for j in range(_JD):
                        x = chunk_v[b, row, pl.ds(j * _L, _L)]
                        plsc.addupdate_scatter(acc_v, [base + (j * _L)], x)
                        plsc.addupdate_scatter(
                            acc_v, [base + (_NSEG * _D + j * _L)], x * x)
                    return carry3

                return lax.fori_loop(0, _L, row_body, 0)

            lax.cond(uniform, fast, slow, 0)
            return carry2

        lax.fori_loop(0, _C // _L, group_body, 0)

    @pl.when(c_lo < c_hi)
    def _():
        start(c_lo, 0)

    def outer(k, carry):
        c = c_lo + 2 * k

        @pl.when(c + 1 < c_hi)
        def _():
            start(c + 1, 1)

        process(c, 0)

        @pl.when(c + 1 < c_hi)
        def _():
            @pl.when(c + 2 < c_hi)
            def _():
                start(c + 2, 0)

            process(c + 1, 1)

        return carry

    npair = (c_hi - c_lo + 1) // 2
    lax.fori_loop(0, npair, outer, 0)

    pltpu.sync_copy(acc_v, out_hbm.at[wid])


_sc_partial = functools.partial(
    pl.kernel,
    out_type=jax.ShapeDtypeStruct((_NW, 2 * _NSEG * _D), jnp.float32),
    mesh=plsc.VectorSubcoreMesh(core_axis_name="c", subcore_axis_name="s"),
    compiler_params=pltpu.CompilerParams(needs_layout_passes=False),
    scratch_types=[
        pltpu.VMEM((2, _C, _D), jnp.float32),
        pltpu.VMEM((2, _C), jnp.int32),
        pltpu.VMEM((2 * _NSEG * _D,), jnp.float32),
        pltpu.SemaphoreType.DMA,
        pltpu.SemaphoreType.DMA,
        pltpu.SemaphoreType.DMA,
        pltpu.SemaphoreType.DMA,
    ],
)(_sc_partial_kernel)


_SEG_PER_TILE = _NSEG // _NW
_W_PER_TILE = _SEG_PER_TILE * _D


def _sc_epilogue_kernel(p_hbm, o_hbm, sum_v, sq_v, out_v):
    cid = lax.axis_index("c")
    sid = lax.axis_index("s")
    wid = sid * 2 + cid
    off = wid * _W_PER_TILE

    pltpu.sync_copy(p_hbm.at[:, pl.ds(off, _W_PER_TILE)], sum_v)
    pltpu.sync_copy(p_hbm.at[:, pl.ds(_NSEG * _D + off, _W_PER_TILE)], sq_v)

    def col(j, carry):
        s = sum_v[0, pl.ds(j * _L, _L)]
        q = sq_v[0, pl.ds(j * _L, _L)]

        def red(t, sq):
            return (sq[0] + sum_v[t, pl.ds(j * _L, _L)],
                    sq[1] + sq_v[t, pl.ds(j * _L, _L)])

        s, q = lax.fori_loop(1, _NW, red, (s, q))
        x = jnp.maximum(q - s * s, 0.0) + _EPS
        i = plsc.bitcast(x, jnp.int32)
        r = plsc.bitcast(jnp.int32(0x5F3759DF) - (i >> 1), jnp.float32)
        for _ in range(3):
            r = r * (1.5 - 0.5 * x * r * r)
        out_v[pl.ds(j * _L, _L)] = x * r
        return carry

    lax.fori_loop(0, _W_PER_TILE // _L, col, 0)
    pltpu.sync_copy(out_v, o_hbm.at[pl.ds(off, _W_PER_TILE)])


_sc_epilogue = functools.partial(
    pl.kernel,
    out_type=jax.ShapeDtypeStruct((_NSEG * _D,), jnp.float32),
    mesh=plsc.VectorSubcoreMesh(core_axis_name="c", subcore_axis_name="s"),
    compiler_params=pltpu.CompilerParams(needs_layout_passes=False),
    scratch_types=[
        pltpu.VMEM((_NW, _W_PER_TILE), jnp.float32),
        pltpu.VMEM((_NW, _W_PER_TILE), jnp.float32),
        pltpu.VMEM((_W_PER_TILE,), jnp.float32),
    ],
)(_sc_epilogue_kernel)


def kernel(feat, segment_ids):
    ids = segment_ids.astype(jnp.int32)
    partials = _sc_partial(feat, ids)
    return _sc_epilogue(partials).reshape(_NSEG, _D)


# speedup vs baseline: 1.0566x; 1.0566x over previous
import functools

import jax
import jax.numpy as jnp
from jax import lax
from jax.experimental import pallas as pl
from jax.experimental.pallas import tpu as pltpu
from jax.experimental.pallas import tpu_sc as plsc

_EPS = 1e-06
_N = 50000
_D = 256
_NSEG = 128
_C = 112
_NCHUNK = (_N + _C - 1) // _C
_L = 16
_NW = 32
_JD = _D // _L


def _sc_partial_kernel(feat_hbm, ids_hbm, out_hbm, chunk_v, idx_v, acc_v,
                       semf0, semf1, semi0, semi1):
    cid = lax.axis_index("c")
    sid = lax.axis_index("s")
    wid = sid * 2 + cid
    semf = (semf0, semf1)
    semi = (semi0, semi1)

    zvec = jnp.zeros((_L,), jnp.float32)
    lane = lax.iota(jnp.int32, _L)

    def zero_row(r, carry):
        for j in range(_JD):
            acc_v[pl.ds(r * _D + j * _L, _L)] = zvec
        return carry

    lax.fori_loop(0, 2 * _NSEG, zero_row, 0)

    c_lo = (wid * _NCHUNK) // _NW
    c_hi = ((wid + 1) * _NCHUNK) // _NW

    def start(c, b):
        off = lax.min(c * _C, _N - _C)
        pltpu.async_copy(feat_hbm.at[pl.ds(off, _C)], chunk_v.at[b], semf[b])
        pltpu.async_copy(ids_hbm.at[pl.ds(off, _C)], idx_v.at[b], semi[b])

    def process(c, b):
        off = lax.min(c * _C, _N - _C)
        nzero = c * _C - off
        pltpu.make_async_copy(feat_hbm.at[pl.ds(off, _C)], chunk_v.at[b],
                              semf[b]).wait()
        pltpu.make_async_copy(ids_hbm.at[pl.ds(off, _C)], idx_v.at[b],
                              semi[b]).wait()

        def zrow(r, carry2):
            for j in range(_JD):
                chunk_v[b, r, pl.ds(j * _L, _L)] = zvec
            return carry2

        lax.fori_loop(0, nzero, zrow, 0)

        def group_body(g, carry2):
            seg_vec = idx_v[b, pl.ds(g * _L, _L)]
            seg0 = seg_vec.at[jnp.zeros((_L,), jnp.int32)].get(
                mode="promise_in_bounds")
            uniform = jnp.all(seg_vec == seg0)
            row0 = g * _L

            def fast(_):
                def racc(r, carry):
                    out = []
                    for j in range(_JD):
                        x = chunk_v[b, row0 + r, pl.ds(j * _L, _L)]
                        out.append(carry[j] + x)
                        out.append(carry[_JD + j] + x * x)
                    return tuple(out[0::2]) + tuple(out[1::2])

                acc = lax.fori_loop(0, _L, racc, (zvec,) * (2 * _JD))
                base = seg0 * _D + lane
                for j in range(_JD):
                    plsc.addupdate_scatter(acc_v, [base + (j * _L)], acc[j])
                    plsc.addupdate_scatter(
                        acc_v, [base + (_NSEG * _D + j * _L)], acc[_JD + j])
                return 0

            def slow(_):
                def row_body(r, carry3):
                    ridx = jnp.full((_L,), r, jnp.int32)
                    seg = seg_vec.at[ridx].get(mode="promise_in_bounds")
                    base = seg * _D + lane
                    row = row0 + r
                    for j in range(_JD):
                        x = chunk_v[b, row, pl.ds(j * _L, _L)]
                        plsc.addupdate_scatter(acc_v, [base + (j * _L)], x)
                        plsc.addupdate_scatter(
                            acc_v, [base + (_NSEG * _D + j * _L)], x * x)
                    return carry3

                return lax.fori_loop(0, _L, row_body, 0)

            lax.cond(uniform, fast, slow, 0)
            return carry2

        lax.fori_loop(0, _C // _L, group_body, 0)

    @pl.when(c_lo < c_hi)
    def _():
        start(c_lo, 0)

    def outer(k, carry):
        c = c_lo + 2 * k

        @pl.when(c + 1 < c_hi)
        def _():
            start(c + 1, 1)

        process(c, 0)

        @pl.when(c + 1 < c_hi)
        def _():
            @pl.when(c + 2 < c_hi)
            def _():
                start(c + 2, 0)

            process(c + 1, 1)

        return carry

    npair = (c_hi - c_lo + 1) // 2
    lax.fori_loop(0, npair, outer, 0)

    pltpu.sync_copy(acc_v, out_hbm.at[wid])


_sc_partial = functools.partial(
    pl.kernel,
    out_type=jax.ShapeDtypeStruct((_NW, 2 * _NSEG * _D), jnp.float32),
    mesh=plsc.VectorSubcoreMesh(core_axis_name="c", subcore_axis_name="s"),
    compiler_params=pltpu.CompilerParams(needs_layout_passes=False),
    scratch_types=[
        pltpu.VMEM((2, _C, _D), jnp.float32),
        pltpu.VMEM((2, _C), jnp.int32),
        pltpu.VMEM((2 * _NSEG * _D,), jnp.float32),
        pltpu.SemaphoreType.DMA,
        pltpu.SemaphoreType.DMA,
        pltpu.SemaphoreType.DMA,
        pltpu.SemaphoreType.DMA,
    ],
)(_sc_partial_kernel)


_SEG_PER_TILE = _NSEG // _NW
_W_PER_TILE = _SEG_PER_TILE * _D


def _sc_epilogue_kernel(p_hbm, o_hbm, sum_v, sq_v, out_v):
    cid = lax.axis_index("c")
    sid = lax.axis_index("s")
    wid = sid * 2 + cid
    off = wid * _W_PER_TILE

    pltpu.sync_copy(p_hbm.at[:, pl.ds(off, _W_PER_TILE)], sum_v)
    pltpu.sync_copy(p_hbm.at[:, pl.ds(_NSEG * _D + off, _W_PER_TILE)], sq_v)

    def col(j, carry):
        s = sum_v[0, pl.ds(j * _L, _L)]
        q = sq_v[0, pl.ds(j * _L, _L)]
        for t in range(1, _NW):
            s = s + sum_v[t, pl.ds(j * _L, _L)]
            q = q + sq_v[t, pl.ds(j * _L, _L)]
        x = jnp.maximum(q - s * s, 0.0) + _EPS
        i = plsc.bitcast(x, jnp.int32)
        r = plsc.bitcast(jnp.int32(0x5F3759DF) - (i >> 1), jnp.float32)
        for _ in range(3):
            r = r * (1.5 - 0.5 * x * r * r)
        out_v[pl.ds(j * _L, _L)] = x * r
        return carry

    lax.fori_loop(0, _W_PER_TILE // _L, col, 0)
    pltpu.sync_copy(out_v, o_hbm.at[pl.ds(off, _W_PER_TILE)])


_sc_epilogue = functools.partial(
    pl.kernel,
    out_type=jax.ShapeDtypeStruct((_NSEG * _D,), jnp.float32),
    mesh=plsc.VectorSubcoreMesh(core_axis_name="c", subcore_axis_name="s"),
    compiler_params=pltpu.CompilerParams(needs_layout_passes=False),
    scratch_types=[
        pltpu.VMEM((_NW, _W_PER_TILE), jnp.float32),
        pltpu.VMEM((_NW, _W_PER_TILE), jnp.float32),
        pltpu.VMEM((_W_PER_TILE,), jnp.float32),
    ],
)(_sc_epilogue_kernel)


def kernel(feat, segment_ids):
    ids = segment_ids.astype(jnp.int32)
    partials = _sc_partial(feat, ids)
    return _sc_epilogue(partials).reshape(_NSEG, _D)


# speedup vs baseline: 1.2127x; 1.1478x over previous
import functools

import jax
import jax.numpy as jnp
from jax import lax
from jax.experimental import pallas as pl
from jax.experimental.pallas import tpu as pltpu
from jax.experimental.pallas import tpu_sc as plsc

_EPS = 1e-06
_N = 50000
_D = 256
_NSEG = 128
_C = 128
_NCHUNK = (_N + _C - 1) // _C
_L = 16
_NSUB = 16
_NBUF = 3
_DH = _D // 2
_JD = _DH // _L
_SEG_PER_TILE = _NSEG // _NSUB
_RW = _SEG_PER_TILE * _DH
_RP = 256


def _sc_kernel(feat_hbm, ids_hbm, out_hbm, chunk_v, idx_v, acc_v,
               reds_v, redq_v, stage_v, slots,
               semf0, semf1, semf2, semi0, semi1, semi2):
    cid = lax.axis_index("c")
    sid = lax.axis_index("s")
    colbase = cid * _DH
    semf = (semf0, semf1, semf2)
    semi = (semi0, semi1, semi2)

    zvec = jnp.zeros((_L,), jnp.float32)
    lane = lax.iota(jnp.int32, _L)

    def zero_row(r, carry):
        for j in range(_JD):
            acc_v[pl.ds(r * _DH + j * _L, _L)] = zvec
        return carry

    lax.fori_loop(0, 2 * _NSEG, zero_row, 0)

    c_lo = (sid * _NCHUNK) // _NSUB
    c_hi = ((sid + 1) * _NCHUNK) // _NSUB

    def start(c, b):
        off = lax.min(c * _C, _N - _C)
        pltpu.async_copy(feat_hbm.at[pl.ds(off, _C), pl.ds(colbase, _DH)],
                         chunk_v.at[b], semf[b])
        pltpu.async_copy(ids_hbm.at[pl.ds(off, _C)], idx_v.at[b], semi[b])

    def process(c, b):
        off = lax.min(c * _C, _N - _C)
        nzero = c * _C - off
        pltpu.make_async_copy(feat_hbm.at[pl.ds(off, _C),
                                          pl.ds(colbase, _DH)],
                              chunk_v.at[b], semf[b]).wait()
        pltpu.make_async_copy(ids_hbm.at[pl.ds(off, _C)], idx_v.at[b],
                              semi[b]).wait()

        def zrow(r, carry2):
            for j in range(_JD):
                chunk_v[b, r, pl.ds(j * _L, _L)] = zvec
            return carry2

        lax.fori_loop(0, nzero, zrow, 0)

        def group_body(g, carry2):
            seg_vec = idx_v[b, pl.ds(g * _L, _L)]
            seg0 = seg_vec.at[jnp.zeros((_L,), jnp.int32)].get(
                mode="promise_in_bounds")
            uniform = jnp.all(seg_vec == seg0)
            row0 = g * _L

            def fast(_):
                def racc(r, carry):
                    out = []
                    for j in range(_JD):
                        x = chunk_v[b, row0 + r, pl.ds(j * _L, _L)]
                        out.append(carry[j] + x)
                        out.append(carry[_JD + j] + x * x)
                    return tuple(out[0::2]) + tuple(out[1::2])

                acc = lax.fori_loop(0, _L, racc, (zvec,) * (2 * _JD))
                base = seg0 * _DH + lane
                for j in range(_JD):
                    plsc.addupdate_scatter(acc_v, [base + (j * _L)], acc[j])
                    plsc.addupdate_scatter(
                        acc_v, [base + (_NSEG * _DH + j * _L)], acc[_JD + j])
                return 0

            def slow(_):
                def row_body(r, carry3):
                    ridx = jnp.full((_L,), r, jnp.int32)
                    seg = seg_vec.at[ridx].get(mode="promise_in_bounds")
                    base = seg * _DH + lane
                    row = row0 + r
                    for j in range(_JD):
                        x = chunk_v[b, row, pl.ds(j * _L, _L)]
                        plsc.addupdate_scatter(acc_v, [base + (j * _L)], x)
                        plsc.addupdate_scatter(
                            acc_v, [base + (_NSEG * _DH + j * _L)], x * x)
                    return carry3

                return lax.fori_loop(0, _L, row_body, 0)

            lax.cond(uniform, fast, slow, 0)
            return carry2

        lax.fori_loop(0, _C // _L, group_body, 0)

    for b in range(_NBUF - 1):
        @pl.when(c_lo + b < c_hi)
        def _(b=b):
            start(c_lo + b, b)

    def outer(k, carry):
        c = c_lo + _NBUF * k
        for b in range(_NBUF):
            @pl.when(c + b < c_hi)
            def _(b=b):
                @pl.when(c + b + _NBUF - 1 < c_hi)
                def _():
                    start(c + b + _NBUF - 1, (b + _NBUF - 1) % _NBUF)

                process(c + b, b)

        return carry

    nouter = (c_hi - c_lo + _NBUF - 1) // _NBUF
    lax.fori_loop(0, nouter, outer, 0)

    pltpu.sync_copy(acc_v, slots.at[sid])
    plsc.subcore_barrier()

    for p in range(_RW // _RP):
        pltpu.sync_copy(slots.at[:, pl.ds(sid * _RW + p * _RP, _RP)], reds_v)
        pltpu.sync_copy(
            slots.at[:, pl.ds(_NSEG * _DH + sid * _RW + p * _RP, _RP)],
            redq_v)

        def col(j, carry, p=p):
            s = reds_v[0, pl.ds(j * _L, _L)]
            q = redq_v[0, pl.ds(j * _L, _L)]
            for t in range(1, _NSUB):
                s = s + reds_v[t, pl.ds(j * _L, _L)]
                q = q + redq_v[t, pl.ds(j * _L, _L)]
            x = jnp.maximum(q - s * s, 0.0) + _EPS
            i = plsc.bitcast(x, jnp.int32)
            r = plsc.bitcast(jnp.int32(0x5F3759DF) - (i >> 1), jnp.float32)
            for _ in range(3):
                r = r * (1.5 - 0.5 * x * r * r)
            wg = p * _RP + j * _L
            stage_v[wg // _DH, pl.ds(wg % _DH, _L)] = x * r
            return carry

        lax.fori_loop(0, _RP // _L, col, 0)
    pltpu.sync_copy(stage_v, out_hbm.at[pl.ds(sid * _SEG_PER_TILE,
                                              _SEG_PER_TILE),
                                        pl.ds(colbase, _DH)])


_sc_pool = functools.partial(
    pl.kernel,
    out_type=jax.ShapeDtypeStruct((_NSEG, _D), jnp.float32),
    mesh=plsc.VectorSubcoreMesh(core_axis_name="c", subcore_axis_name="s"),
    compiler_params=pltpu.CompilerParams(needs_layout_passes=False),
    scratch_types=[
        pltpu.VMEM((_NBUF, _C, _DH), jnp.float32),
        pltpu.VMEM((_NBUF, _C), jnp.int32),
        pltpu.VMEM((2 * _NSEG * _DH,), jnp.float32),
        pltpu.VMEM((_NSUB, _RP), jnp.float32),
        pltpu.VMEM((_NSUB, _RP), jnp.float32),
        pltpu.VMEM((_SEG_PER_TILE, _DH), jnp.float32),
        pltpu.VMEM_SHARED((_NSUB, 2 * _NSEG * _DH), jnp.float32),
        pltpu.SemaphoreType.DMA,
        pltpu.SemaphoreType.DMA,
        pltpu.SemaphoreType.DMA,
        pltpu.SemaphoreType.DMA,
        pltpu.SemaphoreType.DMA,
        pltpu.SemaphoreType.DMA,
    ],
)(_sc_kernel)


def kernel(feat, segment_ids):
    return _sc_pool(feat, segment_ids.astype(jnp.int32))


# speedup vs baseline: 1.2932x; 1.0664x over previous
import functools

import jax
import jax.numpy as jnp
from jax import lax
from jax.experimental import pallas as pl
from jax.experimental.pallas import tpu as pltpu
from jax.experimental.pallas import tpu_sc as plsc

_EPS = 1e-06
_N = 50000
_D = 256
_NSEG = 128
_C = 128
_NCHUNK = (_N + _C - 1) // _C
_L = 16
_NSUB = 16
_NBUF = 3
_DH = _D // 2
_JD = _DH // _L
_SEG_PER_TILE = _NSEG // _NSUB
_RW = _SEG_PER_TILE * _DH
_RP = 256


def _sc_kernel(feat_hbm, ids_hbm, out_hbm, chunk_v, idx_v, acc_v,
               reds_v, redq_v, stage_v, slots,
               semf0, semf1, semf2, semi0, semi1, semi2):
    cid = lax.axis_index("c")
    sid = lax.axis_index("s")
    colbase = cid * _DH
    semf = (semf0, semf1, semf2)
    semi = (semi0, semi1, semi2)

    zvec = jnp.zeros((_L,), jnp.float32)
    lane = lax.iota(jnp.int32, _L)

    def zero_row(r, carry):
        for j in range(_JD):
            acc_v[pl.ds(r * _DH + j * _L, _L)] = zvec
        return carry

    lax.fori_loop(0, 2 * _NSEG, zero_row, 0)

    c_lo = (sid * _NCHUNK) // _NSUB
    c_hi = ((sid + 1) * _NCHUNK) // _NSUB

    def start(c, b):
        off = lax.min(c * _C, _N - _C)
        pltpu.async_copy(feat_hbm.at[pl.ds(off, _C), pl.ds(colbase, _DH)],
                         chunk_v.at[b], semf[b])
        pltpu.async_copy(ids_hbm.at[pl.ds(off, _C)], idx_v.at[b], semi[b])

    def process(c, b):
        off = lax.min(c * _C, _N - _C)
        nzero = c * _C - off
        pltpu.make_async_copy(feat_hbm.at[pl.ds(off, _C),
                                          pl.ds(colbase, _DH)],
                              chunk_v.at[b], semf[b]).wait()
        pltpu.make_async_copy(ids_hbm.at[pl.ds(off, _C)], idx_v.at[b],
                              semi[b]).wait()

        def zrow(r, carry2):
            for j in range(_JD):
                chunk_v[b, r, pl.ds(j * _L, _L)] = zvec
            return carry2

        lax.fori_loop(0, nzero, zrow, 0)

        def flush(seg0, acc):
            base = seg0 * _DH + lane
            for j in range(_JD):
                plsc.addupdate_scatter(acc_v, [base + (j * _L)], acc[j])
                plsc.addupdate_scatter(
                    acc_v, [base + (_NSEG * _DH + j * _L)], acc[_JD + j])

        def racc_rows(row0, nrows):
            def racc(r, carry):
                out = []
                for j in range(_JD):
                    x = chunk_v[b, row0 + r, pl.ds(j * _L, _L)]
                    out.append(carry[j] + x)
                    out.append(carry[_JD + j] + x * x)
                return tuple(out[0::2]) + tuple(out[1::2])

            return lax.fori_loop(0, nrows, racc, (zvec,) * (2 * _JD))

        head = idx_v[b, pl.ds(0, _L)]
        tail = idx_v[b, pl.ds(_C - _L, _L)]
        segh = head.at[jnp.zeros((_L,), jnp.int32)].get(
            mode="promise_in_bounds")
        segt = tail.at[jnp.full((_L,), _L - 1, jnp.int32)].get(
            mode="promise_in_bounds")

        def chunk_fast(_):
            flush(segh, racc_rows(0, _C))
            return 0

        def group_body(g, carry2):
            seg_vec = idx_v[b, pl.ds(g * _L, _L)]
            seg0 = seg_vec.at[jnp.zeros((_L,), jnp.int32)].get(
                mode="promise_in_bounds")
            seg15 = seg_vec.at[jnp.full((_L,), _L - 1, jnp.int32)].get(
                mode="promise_in_bounds")
            uniform = jnp.all(seg0 == seg15)
            row0 = g * _L

            def fast(_):
                flush(seg0, racc_rows(row0, _L))
                return 0

            def slow(_):
                def row_body(r, carry3):
                    ridx = jnp.full((_L,), r, jnp.int32)
                    seg = seg_vec.at[ridx].get(mode="promise_in_bounds")
                    base = seg * _DH + lane
                    row = row0 + r
                    for j in range(_JD):
                        x = chunk_v[b, row, pl.ds(j * _L, _L)]
                        plsc.addupdate_scatter(acc_v, [base + (j * _L)], x)
                        plsc.addupdate_scatter(
                            acc_v, [base + (_NSEG * _DH + j * _L)], x * x)
                    return carry3

                return lax.fori_loop(0, _L, row_body, 0)

            lax.cond(uniform, fast, slow, 0)
            return carry2

        def chunk_groups(_):
            return lax.fori_loop(0, _C // _L, group_body, 0)

        lax.cond(jnp.all(segh == segt), chunk_fast, chunk_groups, 0)

    for b in range(_NBUF - 1):
        @pl.when(c_lo + b < c_hi)
        def _(b=b):
            start(c_lo + b, b)

    def outer(k, carry):
        c = c_lo + _NBUF * k
        for b in range(_NBUF):
            @pl.when(c + b < c_hi)
            def _(b=b):
                @pl.when(c + b + _NBUF - 1 < c_hi)
                def _():
                    start(c + b + _NBUF - 1, (b + _NBUF - 1) % _NBUF)

                process(c + b, b)

        return carry

    nouter = (c_hi - c_lo + _NBUF - 1) // _NBUF
    lax.fori_loop(0, nouter, outer, 0)

    pltpu.sync_copy(acc_v, slots.at[sid])
    plsc.subcore_barrier()

    for p in range(_RW // _RP):
        pltpu.sync_copy(slots.at[:, pl.ds(sid * _RW + p * _RP, _RP)], reds_v)
        pltpu.sync_copy(
            slots.at[:, pl.ds(_NSEG * _DH + sid * _RW + p * _RP, _RP)],
            redq_v)

        def col(j, carry, p=p):
            s = reds_v[0, pl.ds(j * _L, _L)]
            q = redq_v[0, pl.ds(j * _L, _L)]
            for t in range(1, _NSUB):
                s = s + reds_v[t, pl.ds(j * _L, _L)]
                q = q + redq_v[t, pl.ds(j * _L, _L)]
            x = jnp.maximum(q - s * s, 0.0) + _EPS
            i = plsc.bitcast(x, jnp.int32)
            r = plsc.bitcast(jnp.int32(0x5F3759DF) - (i >> 1), jnp.float32)
            for _ in range(3):
                r = r * (1.5 - 0.5 * x * r * r)
            wg = p * _RP + j * _L
            stage_v[wg // _DH, pl.ds(wg % _DH, _L)] = x * r
            return carry

        lax.fori_loop(0, _RP // _L, col, 0)
    pltpu.sync_copy(stage_v, out_hbm.at[pl.ds(sid * _SEG_PER_TILE,
                                              _SEG_PER_TILE),
                                        pl.ds(colbase, _DH)])


_sc_pool = functools.partial(
    pl.kernel,
    out_type=jax.ShapeDtypeStruct((_NSEG, _D), jnp.float32),
    mesh=plsc.VectorSubcoreMesh(core_axis_name="c", subcore_axis_name="s"),
    compiler_params=pltpu.CompilerParams(needs_layout_passes=False),
    scratch_types=[
        pltpu.VMEM((_NBUF, _C, _DH), jnp.float32),
        pltpu.VMEM((_NBUF, _C), jnp.int32),
        pltpu.VMEM((2 * _NSEG * _DH,), jnp.float32),
        pltpu.VMEM((_NSUB, _RP), jnp.float32),
        pltpu.VMEM((_NSUB, _RP), jnp.float32),
        pltpu.VMEM((_SEG_PER_TILE, _DH), jnp.float32),
        pltpu.VMEM_SHARED((_NSUB, 2 * _NSEG * _DH), jnp.float32),
        pltpu.SemaphoreType.DMA,
        pltpu.SemaphoreType.DMA,
        pltpu.SemaphoreType.DMA,
        pltpu.SemaphoreType.DMA,
        pltpu.SemaphoreType.DMA,
        pltpu.SemaphoreType.DMA,
    ],
)(_sc_kernel)


def kernel(feat, segment_ids):
    return _sc_pool(feat, segment_ids.astype(jnp.int32))
